# baseline (speedup 1.0000x reference)
import functools

import jax
import jax.numpy as jnp
from jax import lax
from jax.experimental import pallas as pl
from jax.experimental.pallas import tpu as pltpu
from jax.experimental.pallas import tpu_sc as plsc

N_NODES = 10000
N_EDGES = 320000
D_FEAT = 128

CHUNK = 64
N_CHUNKS = N_EDGES // CHUNK
NC = 2
NS = 16
NW = NC * NS
NT = (N_CHUNKS + NW - 1) // NW
LANES = 16


def _mu_body(x_ref, o_ref):
    x = x_ref[...]
    o_ref[...] = jnp.exp(-0.5 * x * x)


def _node_mu(feat):
    return pl.pallas_call(
        _mu_body,
        out_shape=jax.ShapeDtypeStruct((N_NODES, D_FEAT), jnp.float32),
        grid=(10,),
        in_specs=[pl.BlockSpec((N_NODES // 10, D_FEAT), lambda i: (i, 0))],
        out_specs=pl.BlockSpec((N_NODES // 10, D_FEAT), lambda i: (i, 0)),
    )(feat)


def _make_row_body(g_ref, o_ref):
    def row_body(e, carry):
        for j in range(D_FEAT // LANES):
            sl = pl.ds(j * LANES, LANES)
            o_ref[e, sl] = jnp.minimum(g_ref[e, sl], g_ref[CHUNK + e, sl])
        return carry

    return row_body


_mesh = plsc.VectorSubcoreMesh(core_axis_name="c", subcore_axis_name="s")


@functools.partial(
    pl.kernel,
    mesh=_mesh,
    out_type=jax.ShapeDtypeStruct((N_EDGES, D_FEAT), jnp.float32),
    scratch_types=[
        pltpu.VMEM((2 * CHUNK,), jnp.int32),
        pltpu.VMEM((2 * CHUNK,), jnp.int32),
        pltpu.VMEM((2 * CHUNK, D_FEAT), jnp.float32),
        pltpu.VMEM((2 * CHUNK, D_FEAT), jnp.float32),
        pltpu.VMEM((CHUNK, D_FEAT), jnp.float32),
        pltpu.VMEM((CHUNK, D_FEAT), jnp.float32),
        pltpu.VMEM_SHARED((N_NODES, D_FEAT), jnp.float32),
        pltpu.SemaphoreType.DMA,
        pltpu.SemaphoreType.DMA,
        pltpu.SemaphoreType.DMA,
        pltpu.SemaphoreType.DMA,
        pltpu.SemaphoreType.DMA,
        pltpu.SemaphoreType.DMA,
    ],
)
def _edge_min_kernel(mu_hbm, comb_hbm, out_hbm,
                     cidx0, cidx1, gbuf0, gbuf1, obuf0, obuf1,
                     mu_sh,
                     si0, si1, sg0, sg1, so0, so1):
    w = lax.axis_index("s") * NC + lax.axis_index("c")

    sub = lax.axis_index("s")
    for p in range(5):
        row0 = pl.multiple_of(sub * 624 + p * (2 * CHUNK), 8)
        pltpu.sync_copy(mu_hbm.at[pl.ds(row0, 2 * CHUNK)], gbuf0)
        pltpu.sync_copy(gbuf0, mu_sh.at[pl.ds(row0, 2 * CHUNK)])
    plsc.subcore_barrier()

    cidx = (cidx0, cidx1)
    gbuf = (gbuf0, gbuf1)
    obuf = (obuf0, obuf1)
    si = (si0, si1)
    sg = (sg0, sg1)
    so = (so0, so1)

    def chunk_of(t):
        return NW * t + w

    def issue_idx(t, b):
        c = chunk_of(t)

        @pl.when(c < N_CHUNKS)
        def _():
            pltpu.async_copy(comb_hbm.at[c], cidx[b], si[b])

    def wait_idx(t, b):
        c = chunk_of(t)

        @pl.when(c < N_CHUNKS)
        def _():
            pltpu.make_async_copy(comb_hbm.at[c], cidx[b], si[b]).wait()

    def issue_gather(t, b):
        c = chunk_of(t)

        @pl.when(c < N_CHUNKS)
        def _():
            pltpu.async_copy(mu_sh.at[cidx[b]], gbuf[b], sg[b])

    def wait_gather(t, b):
        c = chunk_of(t)

        @pl.when(c < N_CHUNKS)
        def _():
            pltpu.make_async_copy(mu_sh.at[cidx[b]], gbuf[b], sg[b]).wait()

    issue_idx(0, 0)
    issue_idx(1, 1)
    wait_idx(0, 0)
    issue_gather(0, 0)

    def body(t2, carry):
        for b in range(2):
            t = 2 * t2 + b
            bn = 1 - b
            c = chunk_of(t)

            @pl.when(t + 1 < NT)
            def _():
                wait_idx(t + 1, bn)
                issue_gather(t + 1, bn)

            wait_gather(t, b)

            @pl.when(t + 2 < NT)
            def _():
                issue_idx(t + 2, b)

            @pl.when((t >= 2) & (chunk_of(t - 2) < N_CHUNKS))
            def _():
                pltpu.make_async_copy(
                    obuf[b], out_hbm.at[pl.ds(0, CHUNK)], so[b]).wait()

            @pl.when(c < N_CHUNKS)
            def _():
                lax.fori_loop(0, CHUNK, _make_row_body(gbuf[b], obuf[b]), 0)
                pltpu.async_copy(
                    obuf[b], out_hbm.at[pl.ds(c * CHUNK, CHUNK)], so[b])
        return carry

    lax.fori_loop(0, NT // 2, body, 0)

    t_last = NT - 1
    wait_gather(t_last, 0)

    @pl.when(chunk_of(t_last - 2) < N_CHUNKS)
    def _():
        pltpu.make_async_copy(obuf[0], out_hbm.at[pl.ds(0, CHUNK)], so[0]).wait()

    @pl.when(chunk_of(t_last) < N_CHUNKS)
    def _():
        lax.fori_loop(0, CHUNK, _make_row_body(gbuf0, obuf0), 0)
        pltpu.async_copy(
            obuf0, out_hbm.at[pl.ds(chunk_of(t_last) * CHUNK, CHUNK)], so0)

    @pl.when(chunk_of(NT - 2) < N_CHUNKS)
    def _():
        pltpu.make_async_copy(obuf[1], out_hbm.at[pl.ds(0, CHUNK)], so[1]).wait()

    @pl.when(chunk_of(NT - 1) < N_CHUNKS)
    def _():
        pltpu.make_async_copy(obuf[0], out_hbm.at[pl.ds(0, CHUNK)], so[0]).wait()


def kernel(feat, edge_index, etypes):
    mu = _node_mu(feat)
    src = edge_index[0].astype(jnp.int32).reshape(N_CHUNKS, CHUNK)
    dst = edge_index[1].astype(jnp.int32).reshape(N_CHUNKS, CHUNK)
    comb = jnp.concatenate([src, dst], axis=1)
    return _edge_min_kernel(mu, comb)


# speedup vs baseline: 1.0122x; 1.0122x over previous
import functools

import jax
import jax.numpy as jnp
from jax import lax
from jax.experimental import pallas as pl
from jax.experimental.pallas import tpu as pltpu
from jax.experimental.pallas import tpu_sc as plsc

N_NODES = 10000
N_EDGES = 320000
D_FEAT = 128

CHUNK = 64
N_CHUNKS = N_EDGES // CHUNK
NC = 2
NS = 16
NW = NC * NS
NT = (N_CHUNKS + NW - 1) // NW
LANES = 16
NB = 3


def _mu_body(x_ref, o_ref):
    x = x_ref[...]
    o_ref[...] = jnp.exp(-0.5 * x * x)


def _node_mu(feat):
    return pl.pallas_call(
        _mu_body,
        out_shape=jax.ShapeDtypeStruct((N_NODES, D_FEAT), jnp.float32),
        grid=(10,),
        in_specs=[pl.BlockSpec((N_NODES // 10, D_FEAT), lambda i: (i, 0))],
        out_specs=pl.BlockSpec((N_NODES // 10, D_FEAT), lambda i: (i, 0)),
    )(feat)


def _make_row_body(a_ref, b_ref):
    def row_body(e, carry):
        for j in range(D_FEAT // LANES):
            sl = pl.ds(j * LANES, LANES)
            a_ref[e, sl] = jnp.minimum(a_ref[e, sl], b_ref[e, sl])
        return carry

    return row_body


_mesh = plsc.VectorSubcoreMesh(core_axis_name="c", subcore_axis_name="s")


@functools.partial(
    pl.kernel,
    mesh=_mesh,
    out_type=jax.ShapeDtypeStruct((N_EDGES, D_FEAT), jnp.float32),
    scratch_types=[
        pltpu.VMEM((CHUNK,), jnp.int32),
        pltpu.VMEM((CHUNK,), jnp.int32),
        pltpu.VMEM((CHUNK,), jnp.int32),
        pltpu.VMEM((CHUNK,), jnp.int32),
        pltpu.VMEM((CHUNK,), jnp.int32),
        pltpu.VMEM((CHUNK,), jnp.int32),
        pltpu.VMEM((CHUNK, D_FEAT), jnp.float32),
        pltpu.VMEM((CHUNK, D_FEAT), jnp.float32),
        pltpu.VMEM((CHUNK, D_FEAT), jnp.float32),
        pltpu.VMEM((CHUNK, D_FEAT), jnp.float32),
        pltpu.VMEM((CHUNK, D_FEAT), jnp.float32),
        pltpu.VMEM((CHUNK, D_FEAT), jnp.float32),
        pltpu.VMEM_SHARED((N_NODES, D_FEAT), jnp.float32),
        pltpu.SemaphoreType.DMA,
        pltpu.SemaphoreType.DMA,
        pltpu.SemaphoreType.DMA,
        pltpu.SemaphoreType.DMA,
        pltpu.SemaphoreType.DMA,
        pltpu.SemaphoreType.DMA,
        pltpu.SemaphoreType.DMA,
        pltpu.SemaphoreType.DMA,
        pltpu.SemaphoreType.DMA,
        pltpu.SemaphoreType.DMA,
        pltpu.SemaphoreType.DMA,
        pltpu.SemaphoreType.DMA,
    ],
)
def _edge_min_kernel(mu_hbm, src_hbm, dst_hbm, out_hbm,
                     sidx0, sidx1, sidx2, didx0, didx1, didx2,
                     bufa0, bufa1, bufa2, bufb0, bufb1, bufb2,
                     mu_sh,
                     si0, si1, si2, sga0, sga1, sga2,
                     sgb0, sgb1, sgb2, so0, so1, so2):
    w = lax.axis_index("s") * NC + lax.axis_index("c")

    sub = lax.axis_index("s")
    for p in range(10):
        row0 = pl.multiple_of(sub * 624 + p * CHUNK, 8)
        pltpu.sync_copy(mu_hbm.at[pl.ds(row0, CHUNK)], bufa0)
        pltpu.sync_copy(bufa0, mu_sh.at[pl.ds(row0, CHUNK)])
    plsc.subcore_barrier()

    sidx = (sidx0, sidx1, sidx2)
    didx = (didx0, didx1, didx2)
    bufa = (bufa0, bufa1, bufa2)
    bufb = (bufb0, bufb1, bufb2)
    si = (si0, si1, si2)
    sga = (sga0, sga1, sga2)
    sgb = (sgb0, sgb1, sgb2)
    so = (so0, so1, so2)

    def chunk_of(t):
        return NW * t + w

    def issue_idx(t, s):
        c = chunk_of(t)

        @pl.when(c < N_CHUNKS)
        def _():
            pltpu.async_copy(src_hbm.at[c], sidx[s], si[s])
            pltpu.async_copy(dst_hbm.at[c], didx[s], si[s])

    def wait_idx(t, s):
        c = chunk_of(t)

        @pl.when(c < N_CHUNKS)
        def _():
            pltpu.make_async_copy(src_hbm.at[c], sidx[s], si[s]).wait()
            pltpu.make_async_copy(dst_hbm.at[c], didx[s], si[s]).wait()

    def issue_gathers(t, s):
        c = chunk_of(t)

        @pl.when(c < N_CHUNKS)
        def _():
            pltpu.async_copy(mu_sh.at[sidx[s]], bufa[s], sga[s])
            pltpu.async_copy(mu_sh.at[didx[s]], bufb[s], sgb[s])

    def wait_gathers(t, s):
        c = chunk_of(t)

        @pl.when(c < N_CHUNKS)
        def _():
            pltpu.make_async_copy(mu_sh.at[sidx[s]], bufa[s], sga[s]).wait()
            pltpu.make_async_copy(mu_sh.at[didx[s]], bufb[s], sgb[s]).wait()

    def wait_out(s):
        pltpu.make_async_copy(bufa[s], out_hbm.at[pl.ds(0, CHUNK)], so[s]).wait()

    issue_idx(0, 0)
    issue_idx(1, 1)
    issue_idx(2, 2)
    wait_idx(0, 0)
    issue_gathers(0, 0)
    wait_idx(1, 1)
    issue_gathers(1, 1)

    def step(t, s):
        s2 = (s + 2) % NB
        c = chunk_of(t)

        wait_idx(t + 2, s2)

        @pl.when(t >= 1)
        def _():
            wait_out(s2)

        issue_gathers(t + 2, s2)

        wait_gathers(t, s)

        issue_idx(t + 3, s)

        @pl.when(c < N_CHUNKS)
        def _():
            lax.fori_loop(0, CHUNK, _make_row_body(bufa[s], bufb[s]), 0)
            pltpu.async_copy(bufa[s], out_hbm.at[pl.ds(c * CHUNK, CHUNK)],
                             so[s])

    def body(g, carry):
        for k in range(NB):
            step(NB * g + k, k)
        return carry

    lax.fori_loop(0, NT // NB, body, 0)

    t_last = NT - 1
    wait_gathers(t_last, 0)

    @pl.when(chunk_of(t_last) < N_CHUNKS)
    def _():
        lax.fori_loop(0, CHUNK, _make_row_body(bufa0, bufb0), 0)
        pltpu.async_copy(bufa0, out_hbm.at[pl.ds(chunk_of(t_last) * CHUNK,
                                                 CHUNK)], so0)

    wait_out(2)

    @pl.when(chunk_of(t_last) < N_CHUNKS)
    def _():
        wait_out(0)


def kernel(feat, edge_index, etypes):
    mu = _node_mu(feat)
    src = edge_index[0].astype(jnp.int32).reshape(N_CHUNKS, CHUNK)
    dst = edge_index[1].astype(jnp.int32).reshape(N_CHUNKS, CHUNK)
    return _edge_min_kernel(mu, src, dst)


# speedup vs baseline: 1.0731x; 1.0601x over previous
import functools

import jax
import jax.numpy as jnp
from jax import lax
from jax.experimental import pallas as pl
from jax.experimental.pallas import tpu as pltpu
from jax.experimental.pallas import tpu_sc as plsc

N_NODES = 10000
N_EDGES = 320000
D_FEAT = 128

CHUNK = 64
N_CHUNKS = N_EDGES // CHUNK
NC = 2
NS = 16
NW = NC * NS
NT = (N_CHUNKS + NW - 1) // NW
LANES = 16


def _mu_body(x_ref, o_ref):
    x = x_ref[...]
    o_ref[...] = jnp.exp(-0.5 * x * x)


def _node_mu(feat):
    return pl.pallas_call(
        _mu_body,
        out_shape=jax.ShapeDtypeStruct((N_NODES, D_FEAT), jnp.float32),
        grid=(10,),
        in_specs=[pl.BlockSpec((N_NODES // 10, D_FEAT), lambda i: (i, 0))],
        out_specs=pl.BlockSpec((N_NODES // 10, D_FEAT), lambda i: (i, 0)),
    )(feat)


def _make_row_body(a_ref, b_ref, o_ref):
    def row_body(e, carry):
        for j in range(D_FEAT // LANES):
            sl = pl.ds(j * LANES, LANES)
            o_ref[e, sl] = jnp.minimum(a_ref[e, sl], b_ref[e, sl])
        return carry

    return row_body


_mesh = plsc.VectorSubcoreMesh(core_axis_name="c", subcore_axis_name="s")


@functools.partial(
    pl.kernel,
    mesh=_mesh,
    out_type=jax.ShapeDtypeStruct((N_EDGES, D_FEAT), jnp.float32),
    scratch_types=[
        pltpu.VMEM((CHUNK,), jnp.int32),
        pltpu.VMEM((CHUNK,), jnp.int32),
        pltpu.VMEM((CHUNK,), jnp.int32),
        pltpu.VMEM((CHUNK,), jnp.int32),
        pltpu.VMEM((CHUNK, D_FEAT), jnp.float32),
        pltpu.VMEM((CHUNK, D_FEAT), jnp.float32),
        pltpu.VMEM((CHUNK, D_FEAT), jnp.float32),
        pltpu.VMEM((CHUNK, D_FEAT), jnp.float32),
        pltpu.VMEM((CHUNK, D_FEAT), jnp.float32),
        pltpu.VMEM((CHUNK, D_FEAT), jnp.float32),
        pltpu.VMEM_SHARED((N_NODES, D_FEAT), jnp.float32),
        pltpu.SemaphoreType.DMA,
        pltpu.SemaphoreType.DMA,
        pltpu.SemaphoreType.DMA,
        pltpu.SemaphoreType.DMA,
        pltpu.SemaphoreType.DMA,
        pltpu.SemaphoreType.DMA,
        pltpu.SemaphoreType.DMA,
        pltpu.SemaphoreType.DMA,
    ],
)
def _edge_min_kernel(mu_hbm, src_hbm, dst_hbm, out_hbm,
                     sidx0, sidx1, didx0, didx1,
                     bufa0, bufa1, bufb0, bufb1, obuf0, obuf1,
                     mu_sh,
                     si0, si1, sga0, sga1, sgb0, sgb1, so0, so1):
    w = lax.axis_index("s") * NC + lax.axis_index("c")

    sub = lax.axis_index("s")
    for p in range(10):
        row0 = pl.multiple_of(sub * 624 + p * CHUNK, 8)
        pltpu.sync_copy(mu_hbm.at[pl.ds(row0, CHUNK)], bufa0)
        pltpu.sync_copy(bufa0, mu_sh.at[pl.ds(row0, CHUNK)])
    plsc.subcore_barrier()
    sidx = (sidx0, sidx1)
    didx = (didx0, didx1)
    bufa = (bufa0, bufa1)
    bufb = (bufb0, bufb1)
    obuf = (obuf0, obuf1)
    si = (si0, si1)
    sga = (sga0, sga1)
    sgb = (sgb0, sgb1)
    so = (so0, so1)

    def chunk_of(t):
        return NW * t + w

    def issue_idx(t, b):
        c = chunk_of(t)

        @pl.when(c < N_CHUNKS)
        def _():
            pltpu.async_copy(src_hbm.at[c], sidx[b], si[b])
            pltpu.async_copy(dst_hbm.at[c], didx[b], si[b])

    def wait_idx(t, b):
        c = chunk_of(t)

        @pl.when(c < N_CHUNKS)
        def _():
            pltpu.make_async_copy(src_hbm.at[c], sidx[b], si[b]).wait()
            pltpu.make_async_copy(dst_hbm.at[c], didx[b], si[b]).wait()

    def issue_gathers(t, b):
        c = chunk_of(t)

        @pl.when(c < N_CHUNKS)
        def _():
            pltpu.async_copy(mu_sh.at[sidx[b]], bufa[b], sga[b])
            pltpu.async_copy(mu_sh.at[didx[b]], bufb[b], sgb[b])

    def wait_gathers(t, b):
        c = chunk_of(t)

        @pl.when(c < N_CHUNKS)
        def _():
            pltpu.make_async_copy(mu_sh.at[sidx[b]], bufa[b], sga[b]).wait()
            pltpu.make_async_copy(mu_sh.at[didx[b]], bufb[b], sgb[b]).wait()

    issue_idx(0, 0)
    issue_idx(1, 1)
    wait_idx(0, 0)
    issue_gathers(0, 0)

    def body(t2, carry):
        for b in range(2):
            t = 2 * t2 + b
            bn = 1 - b
            c = chunk_of(t)

            @pl.when(t + 1 < NT)
            def _():
                wait_idx(t + 1, bn)
                issue_gathers(t + 1, bn)

            wait_gathers(t, b)

            @pl.when(t + 2 < NT)
            def _():
                issue_idx(t + 2, b)

            @pl.when((t >= 2) & (chunk_of(t - 2) < N_CHUNKS))
            def _():
                pltpu.make_async_copy(
                    obuf[b], out_hbm.at[pl.ds(0, CHUNK)], so[b]).wait()

            @pl.when(c < N_CHUNKS)
            def _():
                lax.fori_loop(0, CHUNK, _make_row_body(bufa[b], bufb[b], obuf[b]), 0)
                pltpu.async_copy(
                    obuf[b], out_hbm.at[pl.ds(c * CHUNK, CHUNK)], so[b])
        return carry

    lax.fori_loop(0, NT // 2, body, 0)

    t_last = NT - 1
    wait_gathers(t_last, 0)

    @pl.when(chunk_of(t_last - 2) < N_CHUNKS)
    def _():
        pltpu.make_async_copy(obuf[0], out_hbm.at[pl.ds(0, CHUNK)], so[0]).wait()

    @pl.when(chunk_of(t_last) < N_CHUNKS)
    def _():
        lax.fori_loop(0, CHUNK, _make_row_body(bufa0, bufb0, obuf0), 0)
        pltpu.async_copy(
            obuf0, out_hbm.at[pl.ds(chunk_of(t_last) * CHUNK, CHUNK)], so0)

    @pl.when(chunk_of(NT - 2) < N_CHUNKS)
    def _():
        pltpu.make_async_copy(obuf[1], out_hbm.at[pl.ds(0, CHUNK)], so[1]).wait()

    @pl.when(chunk_of(NT - 1) < N_CHUNKS)
    def _():
        pltpu.make_async_copy(obuf[0], out_hbm.at[pl.ds(0, CHUNK)], so[0]).wait()


def kernel(feat, edge_index, etypes):
    mu = _node_mu(feat)
    src = edge_index[0].astype(jnp.int32).reshape(N_CHUNKS, CHUNK)
    dst = edge_index[1].astype(jnp.int32).reshape(N_CHUNKS, CHUNK)
    return _edge_min_kernel(mu, src, dst)
